# baseline (speedup 1.0000x reference)
import functools

import jax
import jax.numpy as jnp
from jax import lax
from jax.experimental import pallas as pl
from jax.experimental.pallas import tpu as pltpu
from jax.experimental.pallas import tpu_sc as plsc

N_TILES = 16
CH = 512
NQ = 4


def _linear_relu_kernel(x_ref, wt_ref, b_ref, *h_refs):
    acc = jnp.dot(x_ref[...], wt_ref[...], preferred_element_type=jnp.float32)
    h = jnp.maximum(acc + b_ref[...], 0.0)
    dq = h_refs[0].shape[1]
    for q, h_ref in enumerate(h_refs):
        h_ref[...] = h[:, q * dq:(q + 1) * dq]


def _linear_relu(x, wt, b2d):
    n, d_in = x.shape
    d_out = wt.shape[1]
    dq = d_out // NQ
    blk = 1000
    return pl.pallas_call(
        _linear_relu_kernel,
        grid=(n // blk,),
        in_specs=[
            pl.BlockSpec((blk, d_in), lambda i: (i, 0)),
            pl.BlockSpec((d_in, d_out), lambda i: (0, 0)),
            pl.BlockSpec((1, d_out), lambda i: (0, 0)),
        ],
        out_specs=[pl.BlockSpec((blk, dq), lambda i: (i, 0))] * NQ,
        out_shape=[jax.ShapeDtypeStruct((n, dq), jnp.float32)] * NQ,
    )(x, wt, b2d)


NB = 4


def _chunks(total):
    out = [CH] * (total // CH)
    if total % CH:
        out.append(total % CH)
    return out


def _sc_aggregate(hq, src2d, dst2d, zeros):
    n, dq = hq[0].shape
    d_out = NQ * dq
    nchunks = src2d.shape[0]
    cpt = nchunks // N_TILES
    nrounds = cpt // NB
    rq = 128
    rows_per_tile = -(-n // (N_TILES * rq)) * rq
    n_pad = rows_per_tile * N_TILES
    full_sizes = _chunks(rows_per_tile)
    tail_sizes = _chunks(n - (N_TILES - 1) * rows_per_tile)
    mesh = plsc.VectorSubcoreMesh(core_axis_name="c", subcore_axis_name="s")

    @functools.partial(
        pl.kernel,
        mesh=mesh,
        out_type=jax.ShapeDtypeStruct((n, d_out), jnp.float32),
        scratch_types=[
            [pltpu.VMEM((NB, CH), jnp.int32)] * 2,
            [pltpu.VMEM((NB, CH), jnp.int32)] * 2,
            [pltpu.VMEM((CH, dq), jnp.float32)] * NB,
            pltpu.VMEM_SHARED((n_pad, dq), jnp.float32),
            pltpu.VMEM_SHARED((n_pad, dq), jnp.float32),
            [pltpu.SemaphoreType.DMA] * NB,
            [pltpu.SemaphoreType.DMA] * NB,
            [pltpu.SemaphoreType.DMA] * 2,
        ],
        compiler_params=pltpu.CompilerParams(use_tc_tiling_on_sc=False),
    )
    def agg(h0_ref, h1_ref, h2_ref, h3_ref, src_ref, dst_ref, z_ref, o_ref,
            src_seg, dst_seg, rows, acc, hstage, gsem, ssem, isem):
        c = lax.axis_index("c")
        s = lax.axis_index("s")
        base = s * rows_per_tile
        chunk0 = s * cpt
        last = N_TILES - 1

        def istart(r, p):
            o = chunk0 + r * NB
            pltpu.async_copy(src_ref.at[pl.ds(o, NB)], src_seg[p], isem[p])
            pltpu.async_copy(dst_ref.at[pl.ds(o, NB)], dst_seg[p], isem[p])

        def iwait(p):
            pltpu.make_async_copy(src_ref.at[pl.ds(0, NB)], src_seg[p],
                                  isem[p]).wait()
            pltpu.make_async_copy(dst_ref.at[pl.ds(0, NB)], dst_seg[p],
                                  isem[p]).wait()

        def run_pass(h_ref, qcol):
            def stage(sizes):
                off = 0
                for k, sz in enumerate(sizes):
                    pltpu.sync_copy(h_ref.at[pl.ds(base + off, sz)],
                                    rows[k].at[pl.ds(0, sz)])
                    pltpu.async_copy(rows[k].at[pl.ds(0, sz)],
                                     hstage.at[pl.ds(base + off, sz)],
                                     ssem[k])
                    off += sz
                for k, sz in enumerate(sizes):
                    pltpu.make_async_copy(rows[k].at[pl.ds(0, sz)],
                                          hstage.at[pl.ds(0, sz)],
                                          ssem[k]).wait()

            @pl.when(s < last)
            def _():
                stage(full_sizes)

            @pl.when(s == last)
            def _():
                stage(tail_sizes)

            pltpu.sync_copy(z_ref, rows[0])
            off = 0
            for sz in full_sizes:
                pltpu.sync_copy(rows[0].at[pl.ds(0, sz)],
                                acc.at[pl.ds(base + off, sz)])
                off += sz
            plsc.subcore_barrier()

            def gwait(b):
                pltpu.make_async_copy(hstage.at[pl.ds(0, CH)], rows[b],
                                      gsem[b]).wait()

            def swait(b):
                pltpu.make_async_copy(rows[b], acc.at[pl.ds(0, CH)],
                                      ssem[b]).wait()

            def gstart(b, p):
                pltpu.async_copy(hstage.at[src_seg[p].at[b]], rows[b],
                                 gsem[b])

            def sstart(b, p):
                pltpu.async_copy(rows[b], acc.at[dst_seg[p].at[b]],
                                 ssem[b], add=True)

            def round_core(r, p, load_next):
                for b in range(NB):
                    gwait(b)
                    sstart(b, p)
                iwait(1 - p)
                for b in range(NB):
                    swait(b)
                    gstart(b, 1 - p)
                if load_next:
                    istart(r + 2, p)

            istart(0, 0)
            iwait(0)
            istart(1, 1)
            for b in range(NB):
                gstart(b, 0)

            def pair_body(r2, carry):
                round_core(2 * r2, 0, True)
                round_core(2 * r2 + 1, 1, True)
                return carry

            lax.fori_loop(0, (nrounds - 2) // 2, pair_body, 0)
            round_core(nrounds - 2, 0, False)
            for b in range(NB):
                gwait(b)
                sstart(b, 1)
            for b in range(NB):
                swait(b)

            plsc.subcore_barrier()

            def writeback(sizes):
                off = 0
                for k, sz in enumerate(sizes):
                    b = k % NB
                    pltpu.sync_copy(acc.at[pl.ds(base + off, sz)],
                                    rows[b].at[pl.ds(0, sz)])
                    pltpu.async_copy(
                        rows[b].at[pl.ds(0, sz)],
                        o_ref.at[pl.ds(base + off, sz), pl.ds(qcol, dq)],
                        ssem[b])
                    off += sz
                for k, sz in enumerate(sizes):
                    pltpu.make_async_copy(
                        rows[k % NB].at[pl.ds(0, sz)],
                        o_ref.at[pl.ds(0, sz), pl.ds(qcol, dq)],
                        ssem[k % NB]).wait()

            @pl.when(s < last)
            def _():
                writeback(full_sizes)

            @pl.when(s == last)
            def _():
                writeback(tail_sizes)

        @pl.when(c == 0)
        def _():
            run_pass(h0_ref, 0)
            run_pass(h1_ref, dq)

        @pl.when(c == 1)
        def _():
            run_pass(h2_ref, 2 * dq)
            run_pass(h3_ref, 3 * dq)

    return agg(*hq, src2d, dst2d, zeros)


def kernel(x, edge_index, W, b):
    n = x.shape[0]
    n_edges = edge_index.shape[1]
    hq = _linear_relu(x, W.T, b.reshape(1, -1))
    quantum = N_TILES * CH * NB * 2
    e_pad = -(-n_edges // quantum) * quantum
    npad = e_pad - n_edges
    src = jnp.concatenate(
        [edge_index[0], jnp.zeros((npad,), jnp.int32)])
    dst = jnp.concatenate(
        [edge_index[1], jnp.full((npad,), n, jnp.int32)])
    src2d = src.reshape(e_pad // CH, CH)
    dst2d = dst.reshape(e_pad // CH, CH)
    zeros = jnp.zeros((CH, hq[0].shape[1]), jnp.float32)
    return _sc_aggregate(hq, src2d, dst2d, zeros)


# speedup vs baseline: 1.1393x; 1.1393x over previous
import functools

import jax
import jax.numpy as jnp
from jax import lax
from jax.experimental import pallas as pl
from jax.experimental.pallas import tpu as pltpu
from jax.experimental.pallas import tpu_sc as plsc

N_TILES = 16
CH = 256
NQ = 4


def _linear_relu_kernel(x_ref, wt_ref, b_ref, *h_refs):
    acc = jnp.dot(x_ref[...], wt_ref[...], preferred_element_type=jnp.float32)
    h = jnp.maximum(acc + b_ref[...], 0.0)
    dq = h_refs[0].shape[1]
    for q, h_ref in enumerate(h_refs):
        h_ref[...] = h[:, q * dq:(q + 1) * dq]


def _linear_relu(x, wt, b2d):
    n, d_in = x.shape
    d_out = wt.shape[1]
    dq = d_out // NQ
    blk = 1000
    return pl.pallas_call(
        _linear_relu_kernel,
        grid=(n // blk,),
        in_specs=[
            pl.BlockSpec((blk, d_in), lambda i: (i, 0)),
            pl.BlockSpec((d_in, d_out), lambda i: (0, 0)),
            pl.BlockSpec((1, d_out), lambda i: (0, 0)),
        ],
        out_specs=[pl.BlockSpec((blk, dq), lambda i: (i, 0))] * NQ,
        out_shape=[jax.ShapeDtypeStruct((n, dq), jnp.float32)] * NQ,
    )(x, wt, b2d)


NB = 8


def _chunks(total):
    out = [CH] * (total // CH)
    if total % CH:
        out.append(total % CH)
    return out


def _sc_aggregate(hq, src2d, dst2d, zeros):
    n, dq = hq[0].shape
    d_out = NQ * dq
    nchunks = src2d.shape[0]
    cpt = nchunks // N_TILES
    nrounds = cpt // NB
    rq = 128
    rows_per_tile = -(-n // (N_TILES * rq)) * rq
    n_pad = rows_per_tile * N_TILES
    full_sizes = _chunks(rows_per_tile)
    tail_sizes = _chunks(n - (N_TILES - 1) * rows_per_tile)
    mesh = plsc.VectorSubcoreMesh(core_axis_name="c", subcore_axis_name="s")

    @functools.partial(
        pl.kernel,
        mesh=mesh,
        out_type=jax.ShapeDtypeStruct((n, d_out), jnp.float32),
        scratch_types=[
            [pltpu.VMEM((NB, CH), jnp.int32)] * 2,
            [pltpu.VMEM((NB, CH), jnp.int32)] * 2,
            [pltpu.VMEM((CH, dq), jnp.float32)] * NB,
            pltpu.VMEM_SHARED((n_pad, dq), jnp.float32),
            pltpu.VMEM_SHARED((n_pad, dq), jnp.float32),
            [pltpu.SemaphoreType.DMA] * NB,
            [pltpu.SemaphoreType.DMA] * NB,
            [pltpu.SemaphoreType.DMA] * 2,
        ],
        compiler_params=pltpu.CompilerParams(use_tc_tiling_on_sc=False),
    )
    def agg(h0_ref, h1_ref, h2_ref, h3_ref, src_ref, dst_ref, z_ref, o_ref,
            src_seg, dst_seg, rows, acc, hstage, gsem, ssem, isem):
        c = lax.axis_index("c")
        s = lax.axis_index("s")
        base = s * rows_per_tile
        chunk0 = s * cpt
        last = N_TILES - 1

        def istart(r, p):
            o = chunk0 + r * NB
            pltpu.async_copy(src_ref.at[pl.ds(o, NB)], src_seg[p], isem[p])
            pltpu.async_copy(dst_ref.at[pl.ds(o, NB)], dst_seg[p], isem[p])

        def iwait(p):
            pltpu.make_async_copy(src_ref.at[pl.ds(0, NB)], src_seg[p],
                                  isem[p]).wait()
            pltpu.make_async_copy(dst_ref.at[pl.ds(0, NB)], dst_seg[p],
                                  isem[p]).wait()

        def run_pass(h_ref, qcol):
            def stage(sizes):
                off = 0
                for k, sz in enumerate(sizes):
                    pltpu.sync_copy(h_ref.at[pl.ds(base + off, sz)],
                                    rows[k].at[pl.ds(0, sz)])
                    pltpu.async_copy(rows[k].at[pl.ds(0, sz)],
                                     hstage.at[pl.ds(base + off, sz)],
                                     ssem[k])
                    off += sz
                for k, sz in enumerate(sizes):
                    pltpu.make_async_copy(rows[k].at[pl.ds(0, sz)],
                                          hstage.at[pl.ds(0, sz)],
                                          ssem[k]).wait()

            @pl.when(s < last)
            def _():
                stage(full_sizes)

            @pl.when(s == last)
            def _():
                stage(tail_sizes)

            pltpu.sync_copy(z_ref, rows[0])
            off = 0
            for sz in full_sizes:
                pltpu.sync_copy(rows[0].at[pl.ds(0, sz)],
                                acc.at[pl.ds(base + off, sz)])
                off += sz
            plsc.subcore_barrier()

            def gwait(b):
                pltpu.make_async_copy(hstage.at[pl.ds(0, CH)], rows[b],
                                      gsem[b]).wait()

            def swait(b):
                pltpu.make_async_copy(rows[b], acc.at[pl.ds(0, CH)],
                                      ssem[b]).wait()

            def gstart(b, p):
                pltpu.async_copy(hstage.at[src_seg[p].at[b]], rows[b],
                                 gsem[b])

            def sstart(b, p):
                pltpu.async_copy(rows[b], acc.at[dst_seg[p].at[b]],
                                 ssem[b], add=True)

            def round_core(r, p, load_next):
                for b in range(NB):
                    gwait(b)
                    sstart(b, p)
                iwait(1 - p)
                for b in range(NB):
                    swait(b)
                    gstart(b, 1 - p)
                if load_next:
                    istart(r + 2, p)

            istart(0, 0)
            iwait(0)
            istart(1, 1)
            for b in range(NB):
                gstart(b, 0)

            def pair_body(r2, carry):
                round_core(2 * r2, 0, True)
                round_core(2 * r2 + 1, 1, True)
                return carry

            lax.fori_loop(0, (nrounds - 2) // 2, pair_body, 0)
            round_core(nrounds - 2, 0, False)
            for b in range(NB):
                gwait(b)
                sstart(b, 1)
            for b in range(NB):
                swait(b)

            plsc.subcore_barrier()

            def writeback(sizes):
                off = 0
                for k, sz in enumerate(sizes):
                    b = k % NB
                    pltpu.sync_copy(acc.at[pl.ds(base + off, sz)],
                                    rows[b].at[pl.ds(0, sz)])
                    pltpu.async_copy(
                        rows[b].at[pl.ds(0, sz)],
                        o_ref.at[pl.ds(base + off, sz), pl.ds(qcol, dq)],
                        ssem[b])
                    off += sz
                for k, sz in enumerate(sizes):
                    pltpu.make_async_copy(
                        rows[k % NB].at[pl.ds(0, sz)],
                        o_ref.at[pl.ds(0, sz), pl.ds(qcol, dq)],
                        ssem[k % NB]).wait()

            @pl.when(s < last)
            def _():
                writeback(full_sizes)

            @pl.when(s == last)
            def _():
                writeback(tail_sizes)

        @pl.when(c == 0)
        def _():
            run_pass(h0_ref, 0)
            run_pass(h1_ref, dq)

        @pl.when(c == 1)
        def _():
            run_pass(h2_ref, 2 * dq)
            run_pass(h3_ref, 3 * dq)

    return agg(*hq, src2d, dst2d, zeros)


def kernel(x, edge_index, W, b):
    n = x.shape[0]
    n_edges = edge_index.shape[1]
    hq = _linear_relu(x, W.T, b.reshape(1, -1))
    quantum = N_TILES * CH * NB * 2
    e_pad = -(-n_edges // quantum) * quantum
    npad = e_pad - n_edges
    src = jnp.concatenate(
        [edge_index[0], jnp.zeros((npad,), jnp.int32)])
    dst = jnp.concatenate(
        [edge_index[1], jnp.full((npad,), n, jnp.int32)])
    src2d = src.reshape(e_pad // CH, CH)
    dst2d = dst.reshape(e_pad // CH, CH)
    zeros = jnp.zeros((CH, hq[0].shape[1]), jnp.float32)
    return _sc_aggregate(hq, src2d, dst2d, zeros)
